# baseline (speedup 1.0000x reference)
import jax
import jax.numpy as jnp
from jax import lax
from jax.experimental import pallas as pl
from jax.experimental.pallas import tpu as pltpu
from jax.experimental.pallas import tpu_sc as plsc

N = 10000
D = 128
E = 320000
DW1 = 144
NC, NS = 2, 16
NW = NC * NS
EPT = E // NW
K = 80
NCH = EPT // K
IB = 25
NB = NCH // IB
RPT = N // NS


def _sc_agg(dw):
  mesh = plsc.VectorSubcoreMesh(core_axis_name="c", subcore_axis_name="s")

  def body(tabs, siA, diA, siB, diB, zrows, P,
           si, di, bufs, acc, sem0, sem1, semi):
    c = lax.axis_index("c")
    s = lax.axis_index("s")
    w = c * NS + s
    rowbase = s * RPT

    def run_phase(tab, sis, dis, out):
      pltpu.sync_copy(sis.at[pl.ds(w * NCH, IB)], si.at[pl.ds(0, IB)])
      pltpu.sync_copy(dis.at[pl.ds(w * NCH, IB)], di.at[pl.ds(0, IB)])
      pltpu.sync_copy(zrows, acc.at[pl.ds(rowbase, RPT)])
      plsc.subcore_barrier()

      pltpu.async_copy(tab.at[si.at[0]], bufs.at[0], sem0)
      for b in range(NB):
        off = (b % 2) * IB
        noff = ((b + 1) % 2) * IB
        hbase = w * NCH + (b + 1) * IB
        if b + 1 < NB:
          pltpu.async_copy(sis.at[pl.ds(hbase, IB)],
                           si.at[pl.ds(noff, IB)], semi)
          pltpu.async_copy(dis.at[pl.ds(hbase, IB)],
                           di.at[pl.ds(noff, IB)], semi)

        def pair(i, carry):
          g0 = off + 2 * i
          pltpu.async_copy(tab.at[si.at[g0 + 1]], bufs.at[1], sem1)
          pltpu.make_async_copy(tab.at[si.at[g0]], bufs.at[0], sem0).wait()
          pltpu.sync_copy(bufs.at[0], acc.at[di.at[g0]], add=True)

          @pl.when(2 * i + 2 < IB)
          def _():
            pltpu.async_copy(tab.at[si.at[g0 + 2]], bufs.at[0], sem0)

          pltpu.make_async_copy(tab.at[si.at[g0 + 1]], bufs.at[1], sem1).wait()
          pltpu.sync_copy(bufs.at[1], acc.at[di.at[g0 + 1]], add=True)
          return carry

        lax.fori_loop(0, IB // 2, pair, 0)
        tail = off + IB - 1
        pltpu.make_async_copy(tab.at[si.at[tail]], bufs.at[0], sem0).wait()
        pltpu.sync_copy(bufs.at[0], acc.at[di.at[tail]], add=True)
        if b + 1 < NB:
          pltpu.make_async_copy(sis.at[pl.ds(hbase, IB)],
                                si.at[pl.ds(noff, IB)], semi).wait()
          pltpu.make_async_copy(dis.at[pl.ds(hbase, IB)],
                                di.at[pl.ds(noff, IB)], semi).wait()
          pltpu.async_copy(tab.at[si.at[noff]], bufs.at[0], sem0)
      plsc.subcore_barrier()
      pltpu.sync_copy(acc.at[pl.ds(rowbase, RPT)],
                      out.at[c, pl.ds(rowbase, RPT)])

    run_phase(tabs.at[1], siA, diA, P.at[0])
    run_phase(tabs.at[0], siB, diB, P.at[1])

  return pl.kernel(
      body,
      out_type=jax.ShapeDtypeStruct((2, NC, N, dw), jnp.float32),
      mesh=mesh,
      scratch_types=[
          pltpu.VMEM((2 * IB, K), jnp.int32),
          pltpu.VMEM((2 * IB, K), jnp.int32),
          pltpu.VMEM((2, K, dw), jnp.float32),
          pltpu.VMEM_SHARED((N, dw), jnp.float32),
          pltpu.SemaphoreType.DMA,
          pltpu.SemaphoreType.DMA,
          pltpu.SemaphoreType.DMA,
      ],
      compiler_params=pltpu.CompilerParams(use_tc_tiling_on_sc=False),
  )


_BN_SCALE = 1.0 / (1.0 + 1e-5) ** 0.5
BLK = 1000


def _tcB_body(P, xs, W1lT, b1, W1rT, g1, be1, WlT, WrT, hl, hr):
  p = P[0, 0] + P[0, 1]
  feat = p[:, :D]
  deg = p[:, D]
  rdeg = 1.0 / jnp.maximum(deg, 1.0)
  agg = feat * rdeg[:, None]
  xd = xs[0][:, :D]
  h = (jnp.dot(agg, W1lT[0], preferred_element_type=jnp.float32)
       + b1[0]
       + jnp.dot(xd, W1rT[0], preferred_element_type=jnp.float32))
  h = h * (g1[...] * _BN_SCALE) + be1[...]
  h = jnp.where(h >= 0.0, h, 0.01 * h)
  hl[0] = jnp.concatenate(
      [jnp.dot(h, WlT[0], preferred_element_type=jnp.float32),
       jnp.zeros((h.shape[0], DW1 - D), jnp.float32)], axis=1)
  hr[0] = jnp.dot(h, WrT[0], preferred_element_type=jnp.float32)


def _tcB(P, xs, W1lT, b1, W1rT, g1, be1, WlT, WrT):
  wspec = pl.BlockSpec((1, D, D), lambda j, i: (j, 0, 0))
  bspec = pl.BlockSpec((1, 1, D), lambda j, i: (j, 0, 0))
  vspec = pl.BlockSpec((1, D), lambda j, i: (0, 0))
  return pl.pallas_call(
      _tcB_body,
      grid=(2, N // BLK),
      in_specs=[
          pl.BlockSpec((1, 2, BLK, DW1), lambda j, i: (j, 0, i, 0)),
          pl.BlockSpec((1, BLK, DW1), lambda j, i: (j, i, 0)),
          wspec, bspec, wspec, vspec, vspec, wspec, wspec,
      ],
      out_specs=[pl.BlockSpec((1, BLK, DW1), lambda j, i: (j, i, 0)),
                 pl.BlockSpec((1, BLK, D), lambda j, i: (j, i, 0))],
      out_shape=[jax.ShapeDtypeStruct((2, N, DW1), jnp.float32),
                 jax.ShapeDtypeStruct((2, N, D), jnp.float32)],
  )(P, xs, W1lT, b1, W1rT, g1, be1, WlT, WrT)


def _tcD_body(P2, degp, hr, b2, g2, be2, out):
  p2 = (P2[0, 0] + P2[0, 1])[:, :D]
  deg = degp[0].sum(axis=1)
  rdeg = 1.0 / jnp.maximum(deg, 1.0)
  o = p2 * rdeg[:, None] + b2[0] + hr[0]
  out[0] = o * (g2[...] * _BN_SCALE) + be2[...]


def _tcD(P2, degp, hr, b2, g2, be2):
  vspec = pl.BlockSpec((1, D), lambda j, i: (0, 0))
  return pl.pallas_call(
      _tcD_body,
      grid=(2, N // BLK),
      in_specs=[
          pl.BlockSpec((1, 2, BLK, DW1), lambda j, i: (j, 0, i, 0)),
          pl.BlockSpec((1, BLK, 2), lambda j, i: (j, i, 0)),
          pl.BlockSpec((1, BLK, D), lambda j, i: (j, i, 0)),
          pl.BlockSpec((1, 1, D), lambda j, i: (j, 0, 0)),
          vspec, vspec,
      ],
      out_specs=pl.BlockSpec((1, BLK, D), lambda j, i: (j, i, 0)),
      out_shape=jax.ShapeDtypeStruct((2, N, D), jnp.float32),
  )(P2, degp, hr, b2, g2, be2)


_sc_agg_l1 = _sc_agg(DW1)
_sc_agg_l2 = _sc_agg_l1


def kernel(x_user, x_item, edge_index_rates, edge_index_rev_rates,
           W1l_ui, b1_ui, W1r_ui, W1l_iu, b1_iu, W1r_iu, gamma1, beta1,
           W2l_ui, b2_ui, W2r_ui, W2l_iu, b2_iu, W2r_iu, gamma2, beta2):
  f32 = jnp.float32
  xs = jnp.stack([x_item, x_user])
  tail = jnp.zeros((2, N, DW1 - D), f32).at[:, :, 0].set(1.0)
  xs_aug = jnp.concatenate([xs, tail], axis=2)

  srcA = edge_index_rates[0].astype(jnp.int32).reshape(NW * NCH, K)
  dstA = edge_index_rates[1].astype(jnp.int32).reshape(NW * NCH, K)
  srcB = edge_index_rev_rates[0].astype(jnp.int32).reshape(NW * NCH, K)
  dstB = edge_index_rev_rates[1].astype(jnp.int32).reshape(NW * NCH, K)

  zrows = jnp.zeros((RPT, DW1), f32)
  P1 = _sc_agg_l1(xs_aug, srcA, dstA, srcB, dstB, zrows)

  row = lambda v: v.reshape(1, D)
  stk = lambda a, b: jnp.stack([a, b])
  hl, hr = _tcB(P1, xs_aug,
                stk(W1l_ui.T, W1l_iu.T), stk(row(b1_ui), row(b1_iu)),
                stk(W1r_ui.T, W1r_iu.T), row(gamma1), row(beta1),
                stk(W2l_iu.T, W2l_ui.T), stk(W2r_ui.T, W2r_iu.T))

  P2 = _sc_agg_l2(hl, srcA, dstA, srcB, dstB, zrows)

  degp = jnp.transpose(P1[:, :, :, D], (0, 2, 1))
  o = _tcD(P2, degp, hr, stk(row(b2_ui), row(b2_iu)), row(gamma2),
           row(beta2))
  return (o[1], o[0])


# speedup vs baseline: 1.1478x; 1.1478x over previous
import jax
import jax.numpy as jnp
from jax import lax
from jax.experimental import pallas as pl
from jax.experimental.pallas import tpu as pltpu
from jax.experimental.pallas import tpu_sc as plsc

N = 10000
D = 128
E = 320000
DW1 = 144
NC, NS = 2, 16
NW = NC * NS
EPT = E // NW
K = 80
NCH = EPT // K
IB = 25
NB = NCH // IB
RPT = N // NS


def _sc_agg(dw):
  mesh = plsc.VectorSubcoreMesh(core_axis_name="c", subcore_axis_name="s")

  def body(tab, sis, dis, zrows, out,
           si, di, bufs, acc, sem0, sem1, semi):
    c = lax.axis_index("c")
    s = lax.axis_index("s")
    w = c * NS + s
    rowbase = s * RPT

    pltpu.sync_copy(sis.at[pl.ds(w * NCH, IB)], si.at[pl.ds(0, IB)])
    pltpu.sync_copy(dis.at[pl.ds(w * NCH, IB)], di.at[pl.ds(0, IB)])
    pltpu.sync_copy(zrows, acc.at[pl.ds(rowbase, RPT)])
    plsc.subcore_barrier()

    pltpu.async_copy(tab.at[si.at[0]], bufs.at[0], sem0)
    for b in range(NB):
      off = (b % 2) * IB
      noff = ((b + 1) % 2) * IB
      hbase = w * NCH + (b + 1) * IB
      if b + 1 < NB:
        pltpu.async_copy(sis.at[pl.ds(hbase, IB)],
                         si.at[pl.ds(noff, IB)], semi)
        pltpu.async_copy(dis.at[pl.ds(hbase, IB)],
                         di.at[pl.ds(noff, IB)], semi)

      def pair(i, carry):
        g0 = off + 2 * i
        pltpu.async_copy(tab.at[si.at[g0 + 1]], bufs.at[1], sem1)
        pltpu.make_async_copy(tab.at[si.at[g0]], bufs.at[0], sem0).wait()
        pltpu.sync_copy(bufs.at[0], acc.at[di.at[g0]], add=True)

        @pl.when(2 * i + 2 < IB)
        def _():
          pltpu.async_copy(tab.at[si.at[g0 + 2]], bufs.at[0], sem0)

        pltpu.make_async_copy(tab.at[si.at[g0 + 1]], bufs.at[1], sem1).wait()
        pltpu.sync_copy(bufs.at[1], acc.at[di.at[g0 + 1]], add=True)
        return carry

      lax.fori_loop(0, IB // 2, pair, 0)
      tail = off + IB - 1
      pltpu.make_async_copy(tab.at[si.at[tail]], bufs.at[0], sem0).wait()
      pltpu.sync_copy(bufs.at[0], acc.at[di.at[tail]], add=True)
      if b + 1 < NB:
        pltpu.make_async_copy(sis.at[pl.ds(hbase, IB)],
                              si.at[pl.ds(noff, IB)], semi).wait()
        pltpu.make_async_copy(dis.at[pl.ds(hbase, IB)],
                              di.at[pl.ds(noff, IB)], semi).wait()
        pltpu.async_copy(tab.at[si.at[noff]], bufs.at[0], sem0)
    plsc.subcore_barrier()
    pltpu.sync_copy(acc.at[pl.ds(rowbase, RPT)],
                    out.at[c, pl.ds(rowbase, RPT)])

  return pl.kernel(
      body,
      out_type=jax.ShapeDtypeStruct((NC, N, dw), jnp.float32),
      mesh=mesh,
      scratch_types=[
          pltpu.VMEM((2 * IB, K), jnp.int32),
          pltpu.VMEM((2 * IB, K), jnp.int32),
          pltpu.VMEM((2, K, dw), jnp.float32),
          pltpu.VMEM_SHARED((N, dw), jnp.float32),
          pltpu.SemaphoreType.DMA,
          pltpu.SemaphoreType.DMA,
          pltpu.SemaphoreType.DMA,
      ],
      compiler_params=pltpu.CompilerParams(use_tc_tiling_on_sc=False),
  )


_BN_SCALE = 1.0 / (1.0 + 1e-5) ** 0.5
BLK = 1000


def _tcB_body(P, xd, W1lT, b1, W1rT, g1, be1, WlT, WrT, hl, hr):
  p = P[0] + P[1]
  feat = p[:, :D]
  deg = p[:, D]
  rdeg = 1.0 / jnp.maximum(deg, 1.0)
  agg = feat * rdeg[:, None]
  h = (jnp.dot(agg, W1lT[...], preferred_element_type=jnp.float32)
       + b1[...]
       + jnp.dot(xd[...], W1rT[...], preferred_element_type=jnp.float32))
  h = h * (g1[...] * _BN_SCALE) + be1[...]
  h = jnp.where(h >= 0.0, h, 0.01 * h)
  hl[...] = jnp.concatenate(
      [jnp.dot(h, WlT[...], preferred_element_type=jnp.float32),
       jnp.zeros((h.shape[0], DW1 - D), jnp.float32)], axis=1)
  hr[...] = jnp.dot(h, WrT[...], preferred_element_type=jnp.float32)


def _tcB(P, xd, W1lT, b1, W1rT, g1, be1, WlT, WrT):
  wspec = pl.BlockSpec((D, D), lambda i: (0, 0))
  vspec = pl.BlockSpec((1, D), lambda i: (0, 0))
  return pl.pallas_call(
      _tcB_body,
      grid=(N // BLK,),
      in_specs=[
          pl.BlockSpec((2, BLK, DW1), lambda i: (0, i, 0)),
          pl.BlockSpec((BLK, D), lambda i: (i, 0)),
          wspec, vspec, wspec, vspec, vspec, wspec, wspec,
      ],
      out_specs=[pl.BlockSpec((BLK, DW1), lambda i: (i, 0)),
                 pl.BlockSpec((BLK, D), lambda i: (i, 0))],
      out_shape=[jax.ShapeDtypeStruct((N, DW1), jnp.float32),
                 jax.ShapeDtypeStruct((N, D), jnp.float32)],
  )(P, xd, W1lT, b1, W1rT, g1, be1, WlT, WrT)


def _tcD_body(P2, degp, hr, b2, g2, be2, out):
  p2 = (P2[0] + P2[1])[:, :D]
  deg = degp[...].sum(axis=1)
  rdeg = 1.0 / jnp.maximum(deg, 1.0)
  o = p2 * rdeg[:, None] + b2[...] + hr[...]
  out[...] = o * (g2[...] * _BN_SCALE) + be2[...]


def _tcD(P2, degp, hr, b2, g2, be2):
  vspec = pl.BlockSpec((1, D), lambda i: (0, 0))
  return pl.pallas_call(
      _tcD_body,
      grid=(N // BLK,),
      in_specs=[
          pl.BlockSpec((2, BLK, DW1), lambda i: (0, i, 0)),
          pl.BlockSpec((BLK, 2), lambda i: (i, 0)),
          pl.BlockSpec((BLK, D), lambda i: (i, 0)),
          vspec, vspec, vspec,
      ],
      out_specs=pl.BlockSpec((BLK, D), lambda i: (i, 0)),
      out_shape=jax.ShapeDtypeStruct((N, D), jnp.float32),
  )(P2, degp, hr, b2, g2, be2)


_sc_pass = _sc_agg(DW1)


def kernel(x_user, x_item, edge_index_rates, edge_index_rev_rates,
           W1l_ui, b1_ui, W1r_ui, W1l_iu, b1_iu, W1r_iu, gamma1, beta1,
           W2l_ui, b2_ui, W2r_ui, W2l_iu, b2_iu, W2r_iu, gamma2, beta2):
  f32 = jnp.float32
  ones_col = jnp.ones((N, 1), f32)
  pad = jnp.zeros((N, DW1 - D - 1), f32)
  ta_user = jnp.concatenate([x_user, ones_col, pad], axis=1)
  ta_item = jnp.concatenate([x_item, ones_col, pad], axis=1)

  srcA = edge_index_rates[0].astype(jnp.int32).reshape(NW * NCH, K)
  dstA = edge_index_rates[1].astype(jnp.int32).reshape(NW * NCH, K)
  srcB = edge_index_rev_rates[0].astype(jnp.int32).reshape(NW * NCH, K)
  dstB = edge_index_rev_rates[1].astype(jnp.int32).reshape(NW * NCH, K)

  zrows = jnp.zeros((RPT, DW1), f32)
  P1_item = _sc_pass(ta_user, srcA, dstA, zrows)
  P1_user = _sc_pass(ta_item, srcB, dstB, zrows)

  row = lambda v: v.reshape(1, D)
  hl_item, hr_item = _tcB(P1_item, x_item, W1l_ui.T, row(b1_ui), W1r_ui.T,
                          row(gamma1), row(beta1), W2l_iu.T, W2r_ui.T)
  hl_user, hr_user = _tcB(P1_user, x_user, W1l_iu.T, row(b1_iu), W1r_iu.T,
                          row(gamma1), row(beta1), W2l_ui.T, W2r_iu.T)

  P2_item = _sc_pass(hl_user, srcA, dstA, zrows)
  P2_user = _sc_pass(hl_item, srcB, dstB, zrows)

  deg_item = P1_item[:, :, D].T
  deg_user = P1_user[:, :, D].T
  o_item = _tcD(P2_item, deg_item, hr_item, row(b2_ui), row(gamma2),
                row(beta2))
  o_user = _tcD(P2_user, deg_user, hr_user, row(b2_iu), row(gamma2),
                row(beta2))
  return (o_user, o_item)
